# baseline (speedup 1.0000x reference)
import jax
import jax.numpy as jnp
from jax.experimental import pallas as pl
from jax.experimental.pallas import tpu as pltpu

_Q = 1024
_KB = 512
_SENTINEL = 2.0e17


def _nn_loss_kernel(qt_ref, keys_ref, out_ref, acc_min, acc_s):
    pid = pl.program_id(0)
    nblk = pl.num_programs(0)

    @pl.when(pid == 0)
    def _init():
        acc_min[...] = jnp.full((1, _Q), jnp.inf, jnp.float32)
        acc_s[...] = jnp.zeros((1, _Q), jnp.float32)

    qx = qt_ref[0:1, :]
    qy = qt_ref[1:2, :]
    kx = keys_ref[:, 0:1]
    ky = keys_ref[:, 1:2]
    dx = keys_ref[:, 2:3]
    dy = keys_ref[:, 3:4]

    ox = qx - kx
    oy = qy - ky
    d2 = ox * ox + oy * oy
    s = dx * oy - dy * ox

    blk_min = jnp.min(d2, axis=0, keepdims=True)
    blk_s = jnp.sum(jnp.where(d2 == blk_min, s, 0.0), axis=0,
                    keepdims=True)

    upd = blk_min < acc_min[...]
    acc_s[...] = jnp.where(upd, blk_s, acc_s[...])
    acc_min[...] = jnp.where(upd, blk_min, acc_min[...])

    @pl.when(pid == nblk - 1)
    def _finish():
        dist = jnp.sqrt(jnp.maximum(acc_min[...], 1e-12))
        signed = dist * jnp.sign(acc_s[...])
        a = jnp.maximum(1.0 + signed, 0.0)
        num = jnp.sum(a)
        den = jnp.sum((a > 0).astype(jnp.float32)) + 1e-06
        out_ref[...] = (num / den).reshape(1, 1)


def kernel(traj, roadgraph_xyz, roadgraph_dir):
    k = roadgraph_xyz.shape[0]
    kpad = ((k + _KB - 1) // _KB) * _KB
    pad = kpad - k
    xyz = jnp.pad(roadgraph_xyz, ((0, pad), (0, 0)), constant_values=_SENTINEL)
    dirs = jnp.pad(roadgraph_dir, ((0, pad), (0, 0)))
    keys = jnp.concatenate([xyz, dirs], axis=1)
    qt = traj.T

    nblk = kpad // _KB
    loss = pl.pallas_call(
        _nn_loss_kernel,
        grid=(nblk,),
        in_specs=[
            pl.BlockSpec((2, _Q), lambda i: (0, 0)),
            pl.BlockSpec((_KB, 4), lambda i: (i, 0)),
        ],
        out_specs=pl.BlockSpec((1, 1), lambda i: (0, 0)),
        out_shape=jax.ShapeDtypeStruct((1, 1), jnp.float32),
        scratch_shapes=[
            pltpu.VMEM((1, _Q), jnp.float32),
            pltpu.VMEM((1, _Q), jnp.float32),
        ],
    )(qt, keys)
    return loss[0, 0]
